# baseline (speedup 1.0000x reference)
import functools

import jax
import jax.numpy as jnp
from jax import lax
from jax.experimental import pallas as pl
from jax.experimental.pallas import tpu as pltpu
from jax.experimental.pallas import tpu_sc as plsc

_V = 1000
_D = 1000
_N = 1024 * 50
_NC = 2
_NS = 16
_NW = _NC * _NS
_NSTRIP = _D // 8
_XCH = 2048
_NXCH = _N // _XCH
_G = 256
_NG = _XCH // _G


def _body(x_hbm, tt_hbm, out_hbm, tsl0, tsl1, xq0, xq1, ob0, ob1,
          osem0, osem1, xsem0, xsem1, tsem0, tsem1):
    sid = lax.axis_index("s")
    wid = sid * _NC + lax.axis_index("c")
    nstrip = jnp.where(wid < _NSTRIP - 3 * _NW, 4, 3)

    def drain(ob, osem):
        pltpu.make_async_copy(
            ob, out_hbm.at[pl.ds(0, 8), pl.ds(0, _G)], osem
        ).wait()

    def xwait(xq, xsem):
        pltpu.make_async_copy(x_hbm.at[0, 0], xq, xsem).wait()

    def twait(tsl, tsem):
        pltpu.make_async_copy(tt_hbm.at[0, 0], tsl, tsem).wait()

    pltpu.async_copy(tt_hbm.at[wid, 0], tsl0, tsem0)

    for i, (tsl, tsem), (ntsl, ntsem) in (
        (0, (tsl0, tsem0), (tsl1, tsem1)),
        (1, (tsl1, tsem1), (tsl0, tsem0)),
        (2, (tsl0, tsem0), (tsl1, tsem1)),
        (3, (tsl1, tsem1), (tsl0, tsem0)),
    ):
        @pl.when(i < nstrip)
        def _():
            s = wid + i * _NW
            r0 = pl.multiple_of(s * 8, 8)
            pltpu.async_copy(x_hbm.at[0, 0], xq0, xsem0)
            twait(tsl, tsem)

            @pl.when(i + 1 < nstrip)
            def _():
                pltpu.async_copy(tt_hbm.at[s + _NW, 0], ntsl, ntsem)

            @pl.loop(0, _NXCH, step=2)
            def _(c0):
                for cp, xq, xsem, nxq, nxsem in (
                    (0, xq0, xsem0, xq1, xsem1),
                    (1, xq1, xsem1, xq0, xsem0),
                ):
                    c = c0 + cp

                    @pl.when(c < _NXCH)
                    def _():
                        @pl.when(c + 1 < _NXCH)
                        def _():
                            pltpu.async_copy(x_hbm.at[c + 1, 0], nxq, nxsem)

                        xwait(xq, xsem)

                        @pl.loop(0, _NG, step=2)
                        def _(g0):
                            for p, ob, osem in (
                                (0, ob0, osem0), (1, ob1, osem1),
                            ):
                                g = g0 + p
                                gg = c * _NG + g

                                @pl.when((gg >= 2) | (i > 0))
                                def _():
                                    drain(ob, osem)

                                for b in range(_G // 16):
                                    jo = pl.multiple_of(g * _G + b * 16, 16)
                                    idxv = xq[pl.ds(jo, 16)]
                                    vals = [
                                        plsc.load_gather(tsl, [idxv + d * _V])
                                        for d in range(8)
                                    ]
                                    for d in range(8):
                                        ob[d, pl.ds(b * 16, 16)] = vals[d]

                                col = pl.multiple_of(c * _XCH + g * _G, 128)
                                pltpu.async_copy(
                                    ob,
                                    out_hbm.at[pl.ds(r0, 8), pl.ds(col, _G)],
                                    osem,
                                )

    drain(ob0, osem0)
    drain(ob1, osem1)


@functools.partial(
    pl.kernel,
    out_type=jax.ShapeDtypeStruct((_D, _N), jnp.float32),
    mesh=plsc.VectorSubcoreMesh(core_axis_name="c", subcore_axis_name="s"),
    scratch_types=[
        pltpu.VMEM((8 * _V,), jnp.float32),
        pltpu.VMEM((8 * _V,), jnp.float32),
        pltpu.VMEM((_XCH,), jnp.int32),
        pltpu.VMEM((_XCH,), jnp.int32),
        pltpu.VMEM((8, _G), jnp.float32),
        pltpu.VMEM((8, _G), jnp.float32),
        pltpu.SemaphoreType.DMA,
        pltpu.SemaphoreType.DMA,
        pltpu.SemaphoreType.DMA,
        pltpu.SemaphoreType.DMA,
        pltpu.SemaphoreType.DMA,
        pltpu.SemaphoreType.DMA,
    ],
    compiler_params=pltpu.CompilerParams(needs_layout_passes=False),
)
def _gather(x_hbm, tt_hbm, out_hbm, tsl0, tsl1, xq0, xq1, ob0, ob1,
            osem0, osem1, xsem0, xsem1, tsem0, tsem1):
    _body(x_hbm, tt_hbm, out_hbm, tsl0, tsl1, xq0, xq1, ob0, ob1,
          osem0, osem1, xsem0, xsem1, tsem0, tsem1)


def kernel(X, table):
    xf = X.reshape(_NXCH, 1, _XCH).astype(jnp.int32)
    ttr = jnp.swapaxes(table, 0, 1).reshape(_NSTRIP, 1, 8 * _V)
    out_t = _gather(xf, ttr)
    return out_t.T


# speedup vs baseline: 1.0085x; 1.0085x over previous
import functools

import jax
import jax.numpy as jnp
from jax import lax
from jax.experimental import pallas as pl
from jax.experimental.pallas import tpu as pltpu
from jax.experimental.pallas import tpu_sc as plsc

_V = 1000
_D = 1000
_N = 1024 * 50
_NC = 2
_NS = 16
_NW = _NC * _NS
_NSTRIP = _D // 8
_XCH = 2048
_NXCH = _N // _XCH
_G = 256
_NG = _XCH // _G


def _body(x_hbm, tt_hbm, out_hbm, tsl, xq0, xq1, ob0, ob1,
          osem0, osem1, xsem0, xsem1):
    sid = lax.axis_index("s")
    wid = sid * _NC + lax.axis_index("c")
    nstrip = jnp.where(wid < _NSTRIP - 3 * _NW, 4, 3)

    def drain(ob, osem):
        pltpu.make_async_copy(
            ob, out_hbm.at[pl.ds(0, 8), pl.ds(0, _G)], osem
        ).wait()

    def xwait(xq, xsem):
        pltpu.make_async_copy(x_hbm.at[0, 0], xq, xsem).wait()

    @pl.loop(0, nstrip)
    def _(i):
        s = wid + i * _NW
        pltpu.sync_copy(tt_hbm.at[s, 0], tsl)
        r0 = pl.multiple_of(s * 8, 8)
        pltpu.async_copy(x_hbm.at[0, 0], xq0, xsem0)

        @pl.loop(0, _NXCH, step=2)
        def _(c0):
            for cp, xq, xsem, nxq, nxsem in (
                (0, xq0, xsem0, xq1, xsem1),
                (1, xq1, xsem1, xq0, xsem0),
            ):
                c = c0 + cp

                @pl.when(c < _NXCH)
                def _():
                    @pl.when(c + 1 < _NXCH)
                    def _():
                        pltpu.async_copy(x_hbm.at[c + 1, 0], nxq, nxsem)

                    xwait(xq, xsem)

                    @pl.loop(0, _NG, step=2)
                    def _(g0):
                        for p, ob, osem in ((0, ob0, osem0), (1, ob1, osem1)):
                            g = g0 + p
                            gg = c * _NG + g

                            @pl.when(gg >= 2)
                            def _():
                                drain(ob, osem)

                            for b in range(_G // 16):
                                jo = pl.multiple_of(g * _G + b * 16, 16)
                                idxv = xq[pl.ds(jo, 16)]
                                vals = [
                                    plsc.load_gather(tsl, [idxv + d * _V])
                                    for d in range(8)
                                ]
                                for d in range(8):
                                    ob[d, pl.ds(b * 16, 16)] = vals[d]

                            col = pl.multiple_of(c * _XCH + g * _G, 128)
                            pltpu.async_copy(
                                ob, out_hbm.at[pl.ds(r0, 8), pl.ds(col, _G)],
                                osem,
                            )

        drain(ob0, osem0)
        drain(ob1, osem1)


@functools.partial(
    pl.kernel,
    out_type=jax.ShapeDtypeStruct((_D, _N), jnp.float32),
    mesh=plsc.VectorSubcoreMesh(core_axis_name="c", subcore_axis_name="s"),
    scratch_types=[
        pltpu.VMEM((8 * _V,), jnp.float32),
        pltpu.VMEM((_XCH,), jnp.int32),
        pltpu.VMEM((_XCH,), jnp.int32),
        pltpu.VMEM((8, _G), jnp.float32),
        pltpu.VMEM((8, _G), jnp.float32),
        pltpu.SemaphoreType.DMA,
        pltpu.SemaphoreType.DMA,
        pltpu.SemaphoreType.DMA,
        pltpu.SemaphoreType.DMA,
    ],
    compiler_params=pltpu.CompilerParams(needs_layout_passes=False),
)
def _gather(x_hbm, tt_hbm, out_hbm, tsl, xq0, xq1, ob0, ob1,
            osem0, osem1, xsem0, xsem1):
    _body(x_hbm, tt_hbm, out_hbm, tsl, xq0, xq1, ob0, ob1,
          osem0, osem1, xsem0, xsem1)


def kernel(X, table):
    xf = X.reshape(_NXCH, 1, _XCH).astype(jnp.int32)
    ttr = jnp.swapaxes(table, 0, 1).reshape(_NSTRIP, 1, 8 * _V)
    out_t = _gather(xf, ttr)
    return out_t.T
